# baseline (speedup 1.0000x reference)
import functools

import jax
import jax.numpy as jnp
from jax import lax
from jax.experimental import pallas as pl
from jax.experimental.pallas import tpu as pltpu
from jax.experimental.pallas import tpu_sc as plsc

B = 16384
D = 32
NLEVELS = 5
NC = 2
NS = 16
NW = NC * NS
BPW = B // NW
CHUNK = 128
NCHUNK = BPW // CHUNK


def _make_sc_kernel():
    mesh = plsc.VectorSubcoreMesh(core_axis_name="c", subcore_axis_name="s")

    @functools.partial(
        pl.kernel,
        mesh=mesh,
        out_type=jax.ShapeDtypeStruct((B, NLEVELS * D), jnp.float32),
        compiler_params=pltpu.CompilerParams(use_tc_tiling_on_sc=False),
        scratch_types=[
            pltpu.VMEM((NCHUNK, CHUNK), jnp.int32),
            pltpu.VMEM((BPW, D), jnp.float32),
            pltpu.SemaphoreType.DMA,
        ],
    )
    def k(i0, i1, i2, i3, i4, t0, t1, t2, t3, t4, out, idx_v, rows_v, sem):
        wid = lax.axis_index("s") * NC + lax.axis_index("c")
        base = wid * BPW
        for lvl, (ih, th) in enumerate(
            ((i0, t0), (i1, t1), (i2, t2), (i3, t3), (i4, t4))
        ):
            pltpu.sync_copy(ih.at[wid], idx_v)
            copies = []
            for j in range(NCHUNK):
                copies.append(
                    pltpu.async_copy(
                        th.at[idx_v.at[j]],
                        rows_v.at[pl.ds(j * CHUNK, CHUNK)],
                        sem,
                    )
                )
            for c in copies:
                c.wait()
            pltpu.sync_copy(rows_v, out.at[pl.ds(base, BPW), pl.ds(lvl * D, D)])

    return k


_sc_embed = _make_sc_kernel()


def kernel(division_ids, department_ids, class_ids, subclass_ids, group_ids,
           W_division, W_department, W_class, W_subclass, W_group):
    idxs = [
        ids.reshape(NW, NCHUNK, CHUNK)
        for ids in (division_ids, department_ids, class_ids, subclass_ids,
                    group_ids)
    ]
    return _sc_embed(*idxs, W_division, W_department, W_class, W_subclass,
                     W_group)
